# baseline (speedup 1.0000x reference)
import functools

import jax
import jax.numpy as jnp
from jax.experimental import pallas as pl

_BA = 1000
_NC = 81


def _dense_body(nblocks, conf_ref, ct_ref, lt_ref, ld_ref,
                s_ref, npos_ref, acc_ref):
    i = pl.program_id(0)

    @pl.when(i == 0)
    def _():
        acc_ref[...] = jnp.zeros_like(acc_ref)

    @pl.when(i % nblocks == 0)
    def _():
        npos_ref[...] = jnp.zeros_like(npos_ref)

    x = conf_ref[0]
    ct = ct_ref[0]
    m = jnp.max(x, axis=1, keepdims=True)
    e = jnp.exp(x - m)
    lse = jnp.log(jnp.sum(e, axis=1, keepdims=True)) + m
    iota_c = jax.lax.broadcasted_iota(jnp.int32, x.shape, 1)
    tgt = jnp.sum(jnp.where(iota_c == ct, x, 0.0), axis=1, keepdims=True)
    ce = lse - tgt

    pos = ct > 0
    posf = pos.astype(jnp.float32)
    s_ref[0] = jnp.where(pos, 0.0, ce)

    diff = ld_ref[0] - lt_ref[0]
    ad = jnp.abs(diff)
    sl1 = jnp.where(ad < 1.0, 0.5 * diff * diff, ad - 0.5)
    loss_l_part = jnp.sum(jnp.sum(sl1, axis=1, keepdims=True) * posf)
    pos_ce_part = jnp.sum(ce * posf)
    npos_part = jnp.sum(posf)

    npos_ref[0] += npos_part
    lane = jax.lax.broadcasted_iota(jnp.int32, (1, 128), 1)
    acc_ref[...] += (jnp.where(lane == 0, loss_l_part, 0.0)
                     + jnp.where(lane == 1, pos_ce_part, 0.0))


def _mine_body(num_priors, s_ref, bits_ref, npos_ref, acc_ref,
               out_l_ref, out_c_ref):
    s = s_ref[...]
    bits = bits_ref[...]
    num_pos = npos_ref[:, :1]
    num_neg = jnp.minimum(3 * num_pos.astype(jnp.int32), num_priors - 1)

    def step(t, p):
        cand = p | (1 << (30 - t))
        cnt = jnp.sum((bits >= cand).astype(jnp.int32), axis=1, keepdims=True)
        return jnp.where(cnt >= num_neg, cand, p)

    p0 = jnp.zeros_like(num_neg)
    p = jax.lax.fori_loop(0, 31, step, p0)

    c_gt = jnp.sum((bits > p).astype(jnp.int32), axis=1, keepdims=True)
    sum_gt = jnp.sum(jnp.where(bits > p, s, 0.0), axis=1, keepdims=True)
    v_f = jnp.max(jnp.where(bits == p, s, 0.0), axis=1, keepdims=True)
    extra = jnp.where(num_neg > c_gt,
                      (num_neg - c_gt).astype(jnp.float32) * v_f, 0.0)
    neg_contrib = jnp.sum(sum_gt + extra)

    n_total = jnp.sum(num_pos)
    acc = acc_ref[...]
    lane = jax.lax.broadcasted_iota(jnp.int32, acc.shape, 1)
    loss_l_total = jnp.sum(jnp.where(lane == 0, acc, 0.0))
    pos_ce_total = jnp.sum(jnp.where(lane == 1, acc, 0.0))
    loss_l = loss_l_total / n_total
    loss_c = (pos_ce_total + neg_contrib) / n_total
    out_l_ref[...] = jnp.full(out_l_ref.shape, loss_l)
    out_c_ref[...] = jnp.full(out_c_ref.shape, loss_c)


def kernel(loc_t, loc_data, conf_t, conf_data):
    num, num_priors, nc = conf_data.shape
    nblocks = num_priors // _BA
    grid = num * nblocks

    conf_r = conf_data.reshape(grid, _BA, nc)
    ct_r = conf_t.reshape(grid, _BA, 1)
    lt_r = loc_t.reshape(grid, _BA, 4)
    ld_r = loc_data.reshape(grid, _BA, 4)

    s_out, npos_out, acc_out = pl.pallas_call(
        functools.partial(_dense_body, nblocks),
        grid=(grid,),
        in_specs=[
            pl.BlockSpec((1, _BA, nc), lambda i: (i, 0, 0)),
            pl.BlockSpec((1, _BA, 1), lambda i: (i, 0, 0)),
            pl.BlockSpec((1, _BA, 4), lambda i: (i, 0, 0)),
            pl.BlockSpec((1, _BA, 4), lambda i: (i, 0, 0)),
        ],
        out_specs=[
            pl.BlockSpec((1, _BA, 1), lambda i: (i, 0, 0)),
            pl.BlockSpec((1, 1, 128), lambda i: (i // nblocks, 0, 0)),
            pl.BlockSpec((1, 128), lambda i: (0, 0)),
        ],
        out_shape=[
            jax.ShapeDtypeStruct((grid, _BA, 1), jnp.float32),
            jax.ShapeDtypeStruct((num, 1, 128), jnp.float32),
            jax.ShapeDtypeStruct((1, 128), jnp.float32),
        ],
    )(conf_r, ct_r, lt_r, ld_r)

    s = s_out.reshape(num, num_priors)
    bits = jax.lax.bitcast_convert_type(s, jnp.int32)
    npos = npos_out.reshape(num, 128)

    out_l, out_c = pl.pallas_call(
        functools.partial(_mine_body, num_priors),
        out_shape=[
            jax.ShapeDtypeStruct((1, 128), jnp.float32),
            jax.ShapeDtypeStruct((1, 128), jnp.float32),
        ],
    )(s, bits, npos, acc_out)

    return (out_l[0, 0], out_c[0, 0])


# speedup vs baseline: 5.1453x; 5.1453x over previous
import functools

import jax
import jax.numpy as jnp
from jax.experimental import pallas as pl

_AL = 2048
_NC = 81


def _dense_body(ablocks, num_priors, cd_ref, ct_ref, lt_ref, ld_ref,
                s_ref, npos_ref, acc_ref):
    b = pl.program_id(0)
    a = pl.program_id(1)

    @pl.when(jnp.logical_and(b == 0, a == 0))
    def _():
        acc_ref[...] = jnp.zeros_like(acc_ref)

    @pl.when(a == 0)
    def _():
        npos_ref[...] = jnp.zeros_like(npos_ref)

    x = cd_ref[0]
    ct = ct_ref[0]
    m = jnp.max(x, axis=0, keepdims=True)
    e = jnp.exp(x - m)
    lse = jnp.log(jnp.sum(e, axis=0, keepdims=True)) + m
    sub_iota = jax.lax.broadcasted_iota(jnp.int32, x.shape, 0)
    tgt = jnp.sum(jnp.where(sub_iota == ct, x, 0.0), axis=0, keepdims=True)
    ce = lse - tgt

    pos = ct > 0
    posf = pos.astype(jnp.float32)
    ai = jax.lax.broadcasted_iota(jnp.int32, ct.shape, 1) + a * _AL
    dead = jnp.logical_or(pos, ai >= num_priors)
    s_ref[0] = jnp.where(dead, 0.0, ce)

    diff = ld_ref[0] - lt_ref[0]
    ad = jnp.abs(diff)
    sl1 = jnp.where(ad < 1.0, 0.5 * diff * diff, ad - 0.5)
    loss_l_part = jnp.sum(jnp.sum(sl1, axis=0, keepdims=True) * posf)
    pos_ce_part = jnp.sum(ce * posf)
    npos_part = jnp.sum(posf)

    npos_ref[0] += npos_part
    lane = jax.lax.broadcasted_iota(jnp.int32, (1, 128), 1)
    acc_ref[...] += (jnp.where(lane == 0, loss_l_part, 0.0)
                     + jnp.where(lane == 1, pos_ce_part, 0.0))


def _mine_body(num_priors, s_ref, bits_ref, npos_ref, acc_ref,
               out_l_ref, out_c_ref):
    s = s_ref[...]
    bits = bits_ref[...]
    num_pos = npos_ref[:, :1]
    num_neg = jnp.minimum(3 * num_pos.astype(jnp.int32), num_priors - 1)

    def step(t, p):
        cand = p | (1 << (30 - t))
        cnt = jnp.sum((bits >= cand).astype(jnp.int32), axis=1, keepdims=True)
        return jnp.where(cnt >= num_neg, cand, p)

    p0 = jnp.zeros_like(num_neg)
    p = jax.lax.fori_loop(0, 31, step, p0)

    c_gt = jnp.sum((bits > p).astype(jnp.int32), axis=1, keepdims=True)
    sum_gt = jnp.sum(jnp.where(bits > p, s, 0.0), axis=1, keepdims=True)
    v_f = jnp.max(jnp.where(bits == p, s, 0.0), axis=1, keepdims=True)
    extra = jnp.where(num_neg > c_gt,
                      (num_neg - c_gt).astype(jnp.float32) * v_f, 0.0)
    neg_contrib = jnp.sum(sum_gt + extra)

    n_total = jnp.sum(num_pos)
    acc = acc_ref[...]
    lane = jax.lax.broadcasted_iota(jnp.int32, acc.shape, 1)
    loss_l_total = jnp.sum(jnp.where(lane == 0, acc, 0.0))
    pos_ce_total = jnp.sum(jnp.where(lane == 1, acc, 0.0))
    loss_l = loss_l_total / n_total
    loss_c = (pos_ce_total + neg_contrib) / n_total
    out_l_ref[...] = jnp.full(out_l_ref.shape, loss_l)
    out_c_ref[...] = jnp.full(out_c_ref.shape, loss_c)


def kernel(loc_t, loc_data, conf_t, conf_data):
    num, num_priors, nc = conf_data.shape
    ablocks = -(-num_priors // _AL)
    p_pad = ablocks * _AL
    pad = p_pad - num_priors

    cd = jnp.pad(jnp.transpose(conf_data, (0, 2, 1)), ((0, 0), (0, 0), (0, pad)))
    lt = jnp.pad(jnp.transpose(loc_t, (0, 2, 1)), ((0, 0), (0, 0), (0, pad)))
    ld = jnp.pad(jnp.transpose(loc_data, (0, 2, 1)), ((0, 0), (0, 0), (0, pad)))
    ct = jnp.pad(conf_t, ((0, 0), (0, pad))).reshape(num, 1, p_pad)

    s_out, npos_out, acc_out = pl.pallas_call(
        functools.partial(_dense_body, ablocks, num_priors),
        grid=(num, ablocks),
        in_specs=[
            pl.BlockSpec((1, nc, _AL), lambda b, a: (b, 0, a)),
            pl.BlockSpec((1, 1, _AL), lambda b, a: (b, 0, a)),
            pl.BlockSpec((1, 4, _AL), lambda b, a: (b, 0, a)),
            pl.BlockSpec((1, 4, _AL), lambda b, a: (b, 0, a)),
        ],
        out_specs=[
            pl.BlockSpec((1, 1, _AL), lambda b, a: (b, 0, a)),
            pl.BlockSpec((1, 1, 128), lambda b, a: (b, 0, 0)),
            pl.BlockSpec((1, 128), lambda b, a: (0, 0)),
        ],
        out_shape=[
            jax.ShapeDtypeStruct((num, 1, p_pad), jnp.float32),
            jax.ShapeDtypeStruct((num, 1, 128), jnp.float32),
            jax.ShapeDtypeStruct((1, 128), jnp.float32),
        ],
    )(cd, ct, lt, ld)

    s = s_out.reshape(num, p_pad)
    bits = jax.lax.bitcast_convert_type(s, jnp.int32)
    npos = npos_out.reshape(num, 128)

    out_l, out_c = pl.pallas_call(
        functools.partial(_mine_body, num_priors),
        out_shape=[
            jax.ShapeDtypeStruct((1, 128), jnp.float32),
            jax.ShapeDtypeStruct((1, 128), jnp.float32),
        ],
    )(s, bits, npos, acc_out)

    return (out_l[0, 0], out_c[0, 0])
